# baseline (speedup 1.0000x reference)
import functools

import jax
import jax.numpy as jnp
from jax import lax
from jax.experimental import pallas as pl
from jax.experimental.pallas import tpu as pltpu
from jax.experimental.pallas import tpu_sc as plsc

N = 10000
E = 320000
NW = 32
CHUNK = 512
CH = 20
E_PAD = NW * CH * CHUNK
NPAD = 10112
STRIPE = NPAD // 16
LEAD = 8

_mesh = plsc.VectorSubcoreMesh(core_axis_name="c", subcore_axis_name="s")
_sc_params = pltpu.CompilerParams(use_tc_tiling_on_sc=False)


@functools.partial(
    pl.kernel,
    out_type=jax.ShapeDtypeStruct((2, NPAD, 16), jnp.float32),
    mesh=_mesh,
    scratch_types=[
        pltpu.VMEM((CH, CHUNK), jnp.int32),
        pltpu.VMEM((CHUNK, 16), jnp.float32),
        pltpu.SemaphoreType.DMA,
        pltpu.VMEM_SHARED((NPAD, 16), jnp.float32),
    ],
    compiler_params=_sc_params,
)
def _deg_kernel(dstq_hbm, zeros_hbm, ones_hbm, out_hbm, didx, ones_v, sem, shared):
    c = lax.axis_index("c")
    s = lax.axis_index("s")
    wid = s * 2 + c
    r0 = s * STRIPE
    pltpu.sync_copy(zeros_hbm.at[pl.ds(r0, STRIPE)], shared.at[pl.ds(r0, STRIPE)])
    pltpu.sync_copy(dstq_hbm.at[pl.ds(wid * CH, CH)], didx)
    pltpu.sync_copy(ones_hbm, ones_v)
    plsc.subcore_barrier()

    for j in range(LEAD):
        pltpu.make_async_copy(ones_v, shared.at[didx.at[j]], sem).start(add=True)

    def body(j, carry):
        pltpu.make_async_copy(ones_v, shared.at[didx.at[j]], sem).wait()
        pltpu.make_async_copy(ones_v, shared.at[didx.at[j + LEAD]], sem).start(add=True)
        return carry

    lax.fori_loop(0, CH - LEAD, body, 0)
    for j in range(CH - LEAD, CH):
        pltpu.make_async_copy(ones_v, shared.at[didx.at[j]], sem).wait()
    plsc.subcore_barrier()
    pltpu.sync_copy(shared.at[pl.ds(r0, STRIPE)], out_hbm.at[c, pl.ds(r0, STRIPE)])


F = 32

@functools.partial(
    pl.kernel,
    out_type=jax.ShapeDtypeStruct((2, NPAD, F), jnp.float32),
    mesh=_mesh,
    scratch_types=[
        pltpu.VMEM((CH, CHUNK), jnp.int32),
        pltpu.VMEM((CH, CHUNK), jnp.int32),
        pltpu.VMEM((CHUNK, F), jnp.float32),
        pltpu.VMEM((CHUNK, F), jnp.float32),
        pltpu.VMEM((CHUNK, F), jnp.float32),
        pltpu.VMEM((CHUNK, F), jnp.float32),
        pltpu.SemaphoreType.DMA,
        pltpu.SemaphoreType.DMA,
        pltpu.VMEM_SHARED((NPAD, F), jnp.float32),
        pltpu.VMEM_SHARED((NPAD, F), jnp.float32),
    ],
    compiler_params=_sc_params,
)
def _edge_kernel(g_hbm, srcq_hbm, dstq_hbm, zeros_hbm, out_hbm,
                 sidx, didx, rb0, rb1, rb2, rb3, semg, sems, shared, shared_g):
    c = lax.axis_index("c")
    s = lax.axis_index("s")
    wid = s * 2 + c
    r0 = s * STRIPE
    pltpu.sync_copy(zeros_hbm.at[pl.ds(r0, STRIPE)], shared.at[pl.ds(r0, STRIPE)])
    pltpu.sync_copy(g_hbm.at[pl.ds(r0, STRIPE)], shared_g.at[pl.ds(r0, STRIPE)])
    pltpu.sync_copy(srcq_hbm.at[pl.ds(wid * CH, CH)], sidx)
    pltpu.sync_copy(dstq_hbm.at[pl.ds(wid * CH, CH)], didx)
    plsc.subcore_barrier()

    rows = (rb0, rb1, rb2, rb3)
    for b in range(4):
        pltpu.make_async_copy(shared_g.at[sidx.at[b]], rows[b], semg).start()

    def body(i, carry):
        for b in range(4):
            m = i * 4 + b
            pltpu.make_async_copy(shared_g.at[sidx.at[m]], rows[b], semg).wait()
            pltpu.make_async_copy(rows[b], shared.at[didx.at[m]], sems).start(add=True)
        for b in range(4):
            m = i * 4 + b
            pltpu.make_async_copy(rows[b], shared.at[didx.at[m]], sems).wait()
            pltpu.make_async_copy(shared_g.at[sidx.at[m + 4]], rows[b], semg).start()
        return carry

    lax.fori_loop(0, CH // 4 - 1, body, 0)
    for b in range(4):
        m = CH - 4 + b
        pltpu.make_async_copy(shared_g.at[sidx.at[m]], rows[b], semg).wait()
        pltpu.make_async_copy(rows[b], shared.at[didx.at[m]], sems).start(add=True)
    for b in range(4):
        m = CH - 4 + b
        pltpu.make_async_copy(rows[b], shared.at[didx.at[m]], sems).wait()
    plsc.subcore_barrier()
    pltpu.sync_copy(shared.at[pl.ds(r0, STRIPE)], out_hbm.at[c, pl.ds(r0, STRIPE)])


def _tc_mm1_body(x_ref, w1_ref, h1_ref):
    h1_ref[:] = jnp.dot(x_ref[:], w1_ref[:], preferred_element_type=jnp.float32)


def _tc_scale_body(h1_ref, degp_ref, g1_ref, dinv_ref):
    degp = degp_ref[:]
    deg = degp[0, :N, 0:1] + degp[1, :N, 0:1] + 1.0
    dinv = lax.rsqrt(deg)
    g1_ref[pl.ds(0, N)] = h1_ref[:] * dinv
    g1_ref[pl.ds(N, NPAD - N)] = jnp.zeros((NPAD - N, 32), jnp.float32)
    dinv_ref[:] = dinv


def _tc_c_body(t1p_ref, g1_ref, dinv_ref, b1_ref, u2_ref):
    t1p = t1p_ref[:]
    dinv = dinv_ref[:]
    g1 = g1_ref[pl.ds(0, N)]
    h1 = dinv * (t1p[0, :N] + t1p[1, :N] + g1) + b1_ref[:]
    u2_ref[pl.ds(0, N)] = h1 * dinv
    u2_ref[pl.ds(N, NPAD - N)] = jnp.zeros((NPAD - N, 32), jnp.float32)


def _tc_e_body(t2p_ref, u2_ref, dinv_ref, w2_ref, b2_ref, batch_ref,
               wf1_ref, bf1_ref, wf2_ref, bf2_ref, out_ref):
    t2p = t2p_ref[:]
    dinv = dinv_ref[:]
    m2 = dinv * (t2p[0, :N] + t2p[1, :N] + u2_ref[pl.ds(0, N)])
    h2 = jnp.dot(m2, w2_ref[:], preferred_element_type=jnp.float32) + b2_ref[:]
    ids = lax.broadcasted_iota(jnp.int32, (64, N), 0)
    oh = (ids == batch_ref[:]).astype(jnp.float32)
    sums = jnp.dot(oh, h2, preferred_element_type=jnp.float32)
    cnts = jnp.sum(oh, axis=1, keepdims=True)
    pooled = sums / jnp.maximum(cnts, 1.0)
    z = jnp.maximum(jnp.dot(pooled, wf1_ref[:],
                            preferred_element_type=jnp.float32) + bf1_ref[:], 0.0)
    y = jnp.dot(z, wf2_ref[:], preferred_element_type=jnp.float32) + bf2_ref[:]
    out_ref[:] = jnp.maximum(y, 0.0) + jnp.log1p(jnp.exp(-jnp.abs(y)))


def kernel(x, edge_index, batch, W1, b1, W2, b2, Wf1, bf1, Wf2, bf2):
    src = edge_index[0]
    dst = edge_index[1]
    pad = E_PAD - E
    pr = jnp.arange(pad, dtype=jnp.int32)
    srcq = jnp.concatenate([src, pr % N]).reshape(NW * CH, CHUNK)
    dstq = jnp.concatenate([dst, N + (pr % (NPAD - N))]).reshape(NW * CH, CHUNK)

    zeros16 = jnp.zeros((NPAD, 16), jnp.float32)
    zeros32 = jnp.zeros((NPAD, 32), jnp.float32)
    ones16 = jnp.ones((CHUNK, 16), jnp.float32)

    degp = _deg_kernel(dstq, zeros16, ones16)
    h1 = pl.pallas_call(
        _tc_mm1_body,
        out_shape=jax.ShapeDtypeStruct((N, 32), jnp.float32),
    )(x, W1)

    g1, dinv = pl.pallas_call(
        _tc_scale_body,
        out_shape=(jax.ShapeDtypeStruct((NPAD, 32), jnp.float32),
                   jax.ShapeDtypeStruct((N, 1), jnp.float32)),
    )(h1, degp)

    t1p = _edge_kernel(g1, srcq, dstq, zeros32)

    u2 = pl.pallas_call(
        _tc_c_body,
        out_shape=jax.ShapeDtypeStruct((NPAD, 32), jnp.float32),
    )(t1p, g1, dinv, b1.reshape(1, 32))

    t2p = _edge_kernel(u2, srcq, dstq, zeros32)

    out = pl.pallas_call(
        _tc_e_body,
        out_shape=jax.ShapeDtypeStruct((64, 10), jnp.float32),
    )(t2p, u2, dinv, W2, b2.reshape(1, 64), batch.reshape(1, N),
      Wf1, bf1.reshape(1, 32), Wf2, bf2.reshape(1, 10))

    return out


# speedup vs baseline: 1.1216x; 1.1216x over previous
import functools

import jax
import jax.numpy as jnp
from jax import lax
from jax.experimental import pallas as pl
from jax.experimental.pallas import tpu as pltpu
from jax.experimental.pallas import tpu_sc as plsc

N = 10000
E = 320000
NW = 32
CHUNK = 512
CH = 20
E_PAD = NW * CH * CHUNK
NPAD = 10112
STRIPE = NPAD // 16
LEAD = 8

_mesh = plsc.VectorSubcoreMesh(core_axis_name="c", subcore_axis_name="s")
_sc_params = pltpu.CompilerParams(use_tc_tiling_on_sc=False)


@functools.partial(
    pl.kernel,
    out_type=jax.ShapeDtypeStruct((2, NPAD), jnp.float32),
    mesh=_mesh,
    scratch_types=[
        pltpu.VMEM((CH, CHUNK), jnp.int32),
        pltpu.VMEM((CHUNK,), jnp.float32),
        pltpu.SemaphoreType.DMA,
        pltpu.VMEM_SHARED((NPAD,), jnp.float32),
    ],
    compiler_params=_sc_params,
)
def _deg_kernel(dstq_hbm, zeros_hbm, ones_hbm, out_hbm, didx, ones_v, sem, shared):
    c = lax.axis_index("c")
    s = lax.axis_index("s")
    wid = s * 2 + c
    r0 = s * STRIPE
    pltpu.sync_copy(zeros_hbm.at[pl.ds(r0, STRIPE)], shared.at[pl.ds(r0, STRIPE)])
    pltpu.sync_copy(dstq_hbm.at[pl.ds(wid * CH, CH)], didx)
    pltpu.sync_copy(ones_hbm, ones_v)
    plsc.subcore_barrier()

    for j in range(LEAD):
        pltpu.make_async_copy(ones_v, shared.at[didx.at[j]], sem).start(add=True)

    def body(j, carry):
        pltpu.make_async_copy(ones_v, shared.at[didx.at[j]], sem).wait()
        pltpu.make_async_copy(ones_v, shared.at[didx.at[j + LEAD]], sem).start(add=True)
        return carry

    lax.fori_loop(0, CH - LEAD, body, 0)
    for j in range(CH - LEAD, CH):
        pltpu.make_async_copy(ones_v, shared.at[didx.at[j]], sem).wait()
    plsc.subcore_barrier()
    pltpu.sync_copy(shared.at[pl.ds(r0, STRIPE)], out_hbm.at[c, pl.ds(r0, STRIPE)])


F = 32

@functools.partial(
    pl.kernel,
    out_type=jax.ShapeDtypeStruct((2, NPAD, F), jnp.float32),
    mesh=_mesh,
    scratch_types=[
        pltpu.VMEM((CH, CHUNK), jnp.int32),
        pltpu.VMEM((CH, CHUNK), jnp.int32),
        pltpu.VMEM((CHUNK, F), jnp.float32),
        pltpu.VMEM((CHUNK, F), jnp.float32),
        pltpu.VMEM((CHUNK, F), jnp.float32),
        pltpu.VMEM((CHUNK, F), jnp.float32),
        pltpu.SemaphoreType.DMA,
        pltpu.SemaphoreType.DMA,
        pltpu.VMEM_SHARED((NPAD, F), jnp.float32),
    ],
    compiler_params=_sc_params,
)
def _edge_kernel(g_hbm, srcq_hbm, dstq_hbm, zeros_hbm, out_hbm,
                 sidx, didx, rb0, rb1, rb2, rb3, semg, sems, shared):
    c = lax.axis_index("c")
    s = lax.axis_index("s")
    wid = s * 2 + c
    r0 = s * STRIPE
    pltpu.sync_copy(zeros_hbm.at[pl.ds(r0, STRIPE)], shared.at[pl.ds(r0, STRIPE)])
    pltpu.sync_copy(srcq_hbm.at[pl.ds(wid * CH, CH)], sidx)
    pltpu.sync_copy(dstq_hbm.at[pl.ds(wid * CH, CH)], didx)
    plsc.subcore_barrier()

    rows = (rb0, rb1, rb2, rb3)
    for b in range(4):
        pltpu.make_async_copy(g_hbm.at[sidx.at[b]], rows[b], semg).start()

    def body(i, carry):
        for b in range(4):
            m = i * 4 + b
            pltpu.make_async_copy(g_hbm.at[sidx.at[m]], rows[b], semg).wait()
            pltpu.make_async_copy(rows[b], shared.at[didx.at[m]], sems).start(add=True)
        for b in range(4):
            m = i * 4 + b
            pltpu.make_async_copy(rows[b], shared.at[didx.at[m]], sems).wait()
            pltpu.make_async_copy(g_hbm.at[sidx.at[m + 4]], rows[b], semg).start()
        return carry

    lax.fori_loop(0, CH // 4 - 1, body, 0)
    for b in range(4):
        m = CH - 4 + b
        pltpu.make_async_copy(g_hbm.at[sidx.at[m]], rows[b], semg).wait()
        pltpu.make_async_copy(rows[b], shared.at[didx.at[m]], sems).start(add=True)
    for b in range(4):
        m = CH - 4 + b
        pltpu.make_async_copy(rows[b], shared.at[didx.at[m]], sems).wait()
    plsc.subcore_barrier()
    pltpu.sync_copy(shared.at[pl.ds(r0, STRIPE)], out_hbm.at[c, pl.ds(r0, STRIPE)])


def _tc_scale_body(x_ref, w1_ref, degp_ref, g1_ref, dinv_ref):
    h1 = jnp.dot(x_ref[:], w1_ref[:], preferred_element_type=jnp.float32)
    degp = degp_ref[:]
    deg = (degp[0, :N] + degp[1, :N] + 1.0).reshape(N, 1)
    dinv = lax.rsqrt(deg)
    g1_ref[:] = h1 * dinv
    dinv_ref[:] = dinv


def _tc_c_body(t1p_ref, g1_ref, dinv_ref, b1_ref, u2_ref):
    t1p = t1p_ref[:]
    dinv = dinv_ref[:]
    g1 = g1_ref[:]
    h1 = dinv * (t1p[0, :N] + t1p[1, :N] + g1) + b1_ref[:]
    u2_ref[:] = h1 * dinv


def _tc_e_body(t2p_ref, u2_ref, dinv_ref, w2_ref, b2_ref, batch_ref,
               wf1_ref, bf1_ref, wf2_ref, bf2_ref, out_ref):
    t2p = t2p_ref[:]
    dinv = dinv_ref[:]
    m2 = dinv * (t2p[0, :N] + t2p[1, :N] + u2_ref[pl.ds(0, N)])
    h2 = jnp.dot(m2, w2_ref[:], preferred_element_type=jnp.float32) + b2_ref[:]
    ids = lax.broadcasted_iota(jnp.int32, (64, N), 0)
    oh = (ids == batch_ref[:]).astype(jnp.float32)
    sums = jnp.dot(oh, h2, preferred_element_type=jnp.float32)
    cnts = jnp.sum(oh, axis=1, keepdims=True)
    pooled = sums / jnp.maximum(cnts, 1.0)
    z = jnp.maximum(jnp.dot(pooled, wf1_ref[:],
                            preferred_element_type=jnp.float32) + bf1_ref[:], 0.0)
    y = jnp.dot(z, wf2_ref[:], preferred_element_type=jnp.float32) + bf2_ref[:]
    out_ref[:] = jnp.maximum(y, 0.0) + jnp.log1p(jnp.exp(-jnp.abs(y)))


def kernel(x, edge_index, batch, W1, b1, W2, b2, Wf1, bf1, Wf2, bf2):
    src = edge_index[0]
    dst = edge_index[1]
    pad = E_PAD - E
    pr = jnp.arange(pad, dtype=jnp.int32)
    srcq = jnp.concatenate([src, pr % N]).reshape(NW * CH, CHUNK)
    dstq = jnp.concatenate([dst, N + (pr % (NPAD - N))]).reshape(NW * CH, CHUNK)

    zerosn = jnp.zeros((NPAD,), jnp.float32)
    zeros32 = jnp.zeros((NPAD, 32), jnp.float32)
    onesc = jnp.ones((CHUNK,), jnp.float32)

    degp = _deg_kernel(dstq, zerosn, onesc)

    g1, dinv = pl.pallas_call(
        _tc_scale_body,
        out_shape=(jax.ShapeDtypeStruct((N, 32), jnp.float32),
                   jax.ShapeDtypeStruct((N, 1), jnp.float32)),
    )(x, W1, degp)

    t1p = _edge_kernel(g1, srcq, dstq, zeros32)

    u2 = pl.pallas_call(
        _tc_c_body,
        out_shape=jax.ShapeDtypeStruct((N, 32), jnp.float32),
    )(t1p, g1, dinv, b1.reshape(1, 32))

    t2p = _edge_kernel(u2, srcq, dstq, zeros32)

    out = pl.pallas_call(
        _tc_e_body,
        out_shape=jax.ShapeDtypeStruct((64, 10), jnp.float32),
    )(t2p, u2, dinv, W2, b2.reshape(1, 64), batch.reshape(1, N),
      Wf1, bf1.reshape(1, 32), Wf2, bf2.reshape(1, 10))

    return out


# speedup vs baseline: 1.5653x; 1.3956x over previous
import functools

import jax
import jax.numpy as jnp
from jax import lax
from jax.experimental import pallas as pl
from jax.experimental.pallas import tpu as pltpu
from jax.experimental.pallas import tpu_sc as plsc

N = 10000
E = 320000
NW = 32
CHUNK = 1000
CH = 10
NPAD = 10112
NF = NPAD // 4
STRIPE = NPAD // 16
ZROWS = STRIPE // 8

_mesh = plsc.VectorSubcoreMesh(core_axis_name="c", subcore_axis_name="s")
_sc_params = pltpu.CompilerParams(use_tc_tiling_on_sc=False)


@functools.partial(
    pl.kernel,
    out_type=jax.ShapeDtypeStruct((2, NPAD), jnp.float32),
    mesh=_mesh,
    scratch_types=[
        pltpu.VMEM((CH, CHUNK), jnp.int32),
        pltpu.VMEM((1024,), jnp.float32),
        pltpu.VMEM((640,), jnp.float32),
        pltpu.SemaphoreType.DMA,
        pltpu.SemaphoreType.DMA,
        pltpu.VMEM_SHARED((NPAD,), jnp.float32),
    ],
    compiler_params=_sc_params,
)
def _deg_kernel(dst_hbm, out_hbm, didx, ones_v, zbuf, semi, sem, shared):
    c = lax.axis_index("c")
    s = lax.axis_index("s")
    wid = s * 2 + c
    base = wid * (CH * CHUNK)
    r0 = s * STRIPE
    one16 = jnp.ones((16,), jnp.float32)
    zero16 = jnp.zeros((16,), jnp.float32)
    for k in range(64):
        ones_v[pl.ds(k * 16, 16)] = one16
    for k in range(40):
        zbuf[pl.ds(k * 16, 16)] = zero16
    pltpu.make_async_copy(zbuf.at[pl.ds(0, STRIPE)],
                          shared.at[pl.ds(r0, STRIPE)], sem).start()
    for j in range(CH):
        pltpu.make_async_copy(dst_hbm.at[pl.ds(base + j * CHUNK, CHUNK)],
                              didx.at[j], semi).start()
    pltpu.make_async_copy(zbuf.at[pl.ds(0, STRIPE)],
                          shared.at[pl.ds(r0, STRIPE)], sem).wait()
    for j in range(CH):
        pltpu.make_async_copy(dst_hbm.at[pl.ds(base + j * CHUNK, CHUNK)],
                              didx.at[j], semi).wait()
    plsc.subcore_barrier()

    ones_c = ones_v.at[pl.ds(0, CHUNK)]
    for j in range(CH):
        pltpu.make_async_copy(ones_c, shared.at[didx.at[j]], sem).start(add=True)
    for j in range(CH):
        pltpu.make_async_copy(ones_c, shared.at[didx.at[j]], sem).wait()
    plsc.subcore_barrier()
    pltpu.sync_copy(shared.at[pl.ds(r0, STRIPE)], out_hbm.at[c, pl.ds(r0, STRIPE)])


F = 32

@functools.partial(
    pl.kernel,
    out_type=jax.ShapeDtypeStruct((2, NPAD, F), jnp.float32),
    mesh=_mesh,
    scratch_types=[
        pltpu.VMEM((CH, CHUNK), jnp.int32),
        pltpu.VMEM((CH, CHUNK), jnp.int32),
        pltpu.VMEM((CHUNK, F), jnp.float32),
        pltpu.VMEM((CHUNK, F), jnp.float32),
        pltpu.VMEM((ZROWS, F), jnp.float32),
        pltpu.SemaphoreType.DMA,
        pltpu.SemaphoreType.DMA,
        pltpu.SemaphoreType.DMA,
        pltpu.VMEM_SHARED((NPAD, F), jnp.float32),
    ],
    compiler_params=_sc_params,
)
def _edge_kernel(g_hbm, src_hbm, dst_hbm, out_hbm,
                 sidx, didx, rb0, rb1, zbuf, semi, semg, sems, shared):
    c = lax.axis_index("c")
    s = lax.axis_index("s")
    wid = s * 2 + c
    base = wid * (CH * CHUNK)
    r0 = s * STRIPE
    zero16 = jnp.zeros((16,), jnp.float32)
    for i in range(ZROWS):
        zbuf[i, pl.ds(0, 16)] = zero16
        zbuf[i, pl.ds(16, 16)] = zero16
    for k in range(8):
        pltpu.make_async_copy(zbuf, shared.at[pl.ds(r0 + k * ZROWS, ZROWS)],
                              sems).start()
    for j in range(CH):
        pltpu.make_async_copy(src_hbm.at[pl.ds(base + j * CHUNK, CHUNK)],
                              sidx.at[j], semi).start()
        pltpu.make_async_copy(dst_hbm.at[pl.ds(base + j * CHUNK, CHUNK)],
                              didx.at[j], semi).start()
    for k in range(8):
        pltpu.make_async_copy(zbuf, shared.at[pl.ds(r0 + k * ZROWS, ZROWS)],
                              sems).wait()
    for j in range(CH):
        pltpu.make_async_copy(src_hbm.at[pl.ds(base + j * CHUNK, CHUNK)],
                              sidx.at[j], semi).wait()
        pltpu.make_async_copy(dst_hbm.at[pl.ds(base + j * CHUNK, CHUNK)],
                              didx.at[j], semi).wait()
    plsc.subcore_barrier()

    rows = (rb0, rb1)
    for b in range(2):
        pltpu.make_async_copy(g_hbm.at[sidx.at[b]], rows[b], semg).start()

    def body(i, carry):
        for b in range(2):
            m = i * 2 + b
            pltpu.make_async_copy(g_hbm.at[sidx.at[m]], rows[b], semg).wait()
            pltpu.make_async_copy(rows[b], shared.at[didx.at[m]], sems).start(add=True)
        for b in range(2):
            m = i * 2 + b
            pltpu.make_async_copy(rows[b], shared.at[didx.at[m]], sems).wait()
            pltpu.make_async_copy(g_hbm.at[sidx.at[m + 2]], rows[b], semg).start()
        return carry

    lax.fori_loop(0, CH // 2 - 1, body, 0)
    for b in range(2):
        m = CH - 2 + b
        pltpu.make_async_copy(g_hbm.at[sidx.at[m]], rows[b], semg).wait()
        pltpu.make_async_copy(rows[b], shared.at[didx.at[m]], sems).start(add=True)
    for b in range(2):
        m = CH - 2 + b
        pltpu.make_async_copy(rows[b], shared.at[didx.at[m]], sems).wait()
    plsc.subcore_barrier()
    pltpu.sync_copy(shared.at[pl.ds(r0, STRIPE)], out_hbm.at[c, pl.ds(r0, STRIPE)])


def _k4():
    a = lax.broadcasted_iota(jnp.int32, (4, 128), 0)
    j = lax.broadcasted_iota(jnp.int32, (4, 128), 1)
    return (a == j // 32).astype(jnp.float32)


def _tc_scale_body(x4_ref, w1e_ref, deg4p_ref, g_ref, dinv_ref):
    deg4p = deg4p_ref[:]
    dinv4 = lax.rsqrt(deg4p[0] + deg4p[1] + 1.0)
    dinvflat = jnp.dot(dinv4, _k4(), preferred_element_type=jnp.float32)
    hflat = jnp.dot(x4_ref[:], w1e_ref[:], preferred_element_type=jnp.float32)
    g_ref[:] = hflat * dinvflat
    dinv_ref[:] = dinvflat


def _tc_c_body(t1p_ref, g_ref, dinv_ref, b1f_ref, u2_ref):
    t = t1p_ref[:]
    d = dinv_ref[:]
    u2_ref[:] = d * d * (t[0] + t[1] + g_ref[:]) + d * b1f_ref[:]


def _tc_e_body(t2p_ref, u2_ref, dinv_ref, w2e_ref, b2t_ref, batch4_ref,
               wf1_ref, bf1_ref, wf2_ref, bf2_ref, out_ref):
    t = t2p_ref[:]
    d = dinv_ref[:]
    m2 = d * (t[0] + t[1] + u2_ref[:])
    h24 = jnp.dot(m2, w2e_ref[:], preferred_element_type=jnp.float32) + b2t_ref[:]
    batch4 = batch4_ref[:]
    kidx = lax.broadcasted_iota(jnp.int32, (64, NF), 0)
    sums = jnp.zeros((64, 64), jnp.float32)
    cnts = jnp.zeros((64, 1), jnp.float32)
    for a in range(4):
        oh = (kidx == batch4[a:a + 1, :]).astype(jnp.float32)
        sums = sums + jnp.dot(oh, h24[:, 64 * a:64 * a + 64],
                              preferred_element_type=jnp.float32)
        cnts = cnts + jnp.sum(oh, axis=1, keepdims=True)
    pooled = sums / jnp.maximum(cnts, 1.0)
    z = jnp.maximum(jnp.dot(pooled, wf1_ref[:],
                            preferred_element_type=jnp.float32) + bf1_ref[:], 0.0)
    y = jnp.dot(z, wf2_ref[:], preferred_element_type=jnp.float32) + bf2_ref[:]
    out_ref[:] = jnp.maximum(y, 0.0) + jnp.log1p(jnp.exp(-jnp.abs(y)))


def kernel(x, edge_index, batch, W1, b1, W2, b2, Wf1, bf1, Wf2, bf2):
    src = edge_index[0]
    dst = edge_index[1]
    b1flat = jnp.tile(b1, 4).reshape(1, 128)
    b2tile = jnp.tile(b2, 4).reshape(1, 256)
    W2exp = jnp.kron(jnp.eye(4, dtype=jnp.float32), W2)
    batch4 = jnp.concatenate(
        [batch, jnp.full((NPAD - N,), -1, jnp.int32)]).reshape(NF, 4).T

    x4 = jnp.concatenate(
        [x, jnp.zeros((NPAD - N, 128), jnp.float32)]).reshape(NF, 512)
    W1exp = jnp.kron(jnp.eye(4, dtype=jnp.float32), W1)

    degp = _deg_kernel(dst)

    gflat, dinvflat = pl.pallas_call(
        _tc_scale_body,
        out_shape=(jax.ShapeDtypeStruct((NF, 128), jnp.float32),
                   jax.ShapeDtypeStruct((NF, 128), jnp.float32)),
    )(x4, W1exp, degp.reshape(2, NF, 4))

    t1p = _edge_kernel(gflat.reshape(NPAD, F), src, dst)

    u2flat = pl.pallas_call(
        _tc_c_body,
        out_shape=jax.ShapeDtypeStruct((NF, 128), jnp.float32),
    )(t1p.reshape(2, NF, 128), gflat, dinvflat, b1flat)

    t2p = _edge_kernel(u2flat.reshape(NPAD, F), src, dst)

    out = pl.pallas_call(
        _tc_e_body,
        out_shape=jax.ShapeDtypeStruct((64, 10), jnp.float32),
    )(t2p.reshape(2, NF, 128), u2flat, dinvflat, W2exp, b2tile, batch4,
      Wf1, bf1.reshape(1, 32), Wf2, bf2.reshape(1, 10))

    return out
